# baseline (speedup 1.0000x reference)
import functools

import jax
import jax.numpy as jnp
from jax import lax
from jax.experimental import pallas as pl
from jax.experimental.pallas import tpu as pltpu
from jax.experimental.pallas import tpu_sc as plsc

LATENT = 32
BATCH = 16384

NC = 2
NS = 16
NW = NC * NS
B_PER_W = BATCH // NW
CHUNK = 128
NCHUNK = B_PER_W // CHUNK

MLP_BLK = 2048


def _gather_body(s_tab, a_tab, idx_hbm, out_s, out_a, idx_v, rows_s, rows_a, sem):
    wid = lax.axis_index("s") * NC + lax.axis_index("c")
    base = wid * B_PER_W
    pltpu.sync_copy(idx_hbm.at[0, wid], idx_v.at[0])
    pltpu.sync_copy(idx_hbm.at[1, wid], idx_v.at[1])
    copies = []
    for j in range(NCHUNK):
        copies.append(pltpu.async_copy(
            s_tab.at[idx_v.at[0, j]], rows_s.at[pl.ds(j * CHUNK, CHUNK)], sem))
        copies.append(pltpu.async_copy(
            a_tab.at[idx_v.at[1, j]], rows_a.at[pl.ds(j * CHUNK, CHUNK)], sem))
    for c in copies:
        c.wait()
    pltpu.sync_copy(rows_s, out_s.at[pl.ds(base, B_PER_W)])
    pltpu.sync_copy(rows_a, out_a.at[pl.ds(base, B_PER_W)])


@jax.jit
def _sc_gather(student_table, assessment_table, idx):
    mesh = plsc.VectorSubcoreMesh(core_axis_name="c", subcore_axis_name="s")
    return pl.kernel(
        _gather_body,
        out_type=(
            jax.ShapeDtypeStruct((BATCH, LATENT), jnp.float32),
            jax.ShapeDtypeStruct((BATCH, LATENT), jnp.float32),
        ),
        mesh=mesh,
        scratch_types=[
            pltpu.VMEM((2, NCHUNK, CHUNK), jnp.int32),
            pltpu.VMEM((B_PER_W, LATENT), jnp.float32),
            pltpu.VMEM((B_PER_W, LATENT), jnp.float32),
            pltpu.SemaphoreType.DMA,
        ],
        compiler_params=pltpu.CompilerParams(use_tc_tiling_on_sc=False),
    )(student_table, assessment_table, idx)


def _mlp_body(sv, av, w1a, w1b, b1, w2, b2, w3, b3, out):
    h = jnp.maximum(
        jnp.dot(sv[...], w1a[...], preferred_element_type=jnp.float32)
        + jnp.dot(av[...], w1b[...], preferred_element_type=jnp.float32)
        + b1[...], 0.0)
    h = jnp.maximum(
        jnp.dot(h, w2[...], preferred_element_type=jnp.float32) + b2[...], 0.0)
    out[...] = jnp.sum(h * w3[...], axis=-1, keepdims=True) + b3[...]


@jax.jit
def _tc_mlp(sv, av, w1a, w1b, b1, w2, b2, w3, b3):
    grid = (BATCH // MLP_BLK,)
    full = lambda shape: pl.BlockSpec(shape, lambda i: (0, 0))
    return pl.pallas_call(
        _mlp_body,
        grid=grid,
        in_specs=[
            pl.BlockSpec((MLP_BLK, LATENT), lambda i: (i, 0)),
            pl.BlockSpec((MLP_BLK, LATENT), lambda i: (i, 0)),
            full((LATENT, 64)),
            full((LATENT, 64)),
            full((1, 64)),
            full((64, LATENT)),
            full((1, LATENT)),
            full((1, LATENT)),
            full((1, 1)),
        ],
        out_specs=pl.BlockSpec((MLP_BLK, 1), lambda i: (i, 0)),
        out_shape=jax.ShapeDtypeStruct((BATCH, 1), jnp.float32),
    )(sv, av, w1a, w1b, b1, w2, b2, w3, b3)


def kernel(inputs, student_table, assessment_table, W1, b1, W2, b2, W3, b3):
    idx = inputs.T.reshape(2, NW, NCHUNK, CHUNK)
    sv, av = _sc_gather(student_table, assessment_table, idx)
    return _tc_mlp(
        sv, av,
        W1[:LATENT], W1[LATENT:], b1.reshape(1, 64),
        W2, b2.reshape(1, LATENT),
        W3.reshape(1, LATENT), b3.reshape(1, 1),
    )
